# baseline (speedup 1.0000x reference)
import functools
import math

import jax
import jax.numpy as jnp
from jax import lax
from jax.experimental import pallas as pl
from jax.experimental.pallas import tpu as pltpu
from jax.experimental.pallas import tpu_sc as plsc

NC, NS, CH = 2, 16, 128
NW = NC * NS


def _sc_mesh():
    return plsc.VectorSubcoreMesh(
        core_axis_name="c", subcore_axis_name="s", num_cores=NC, num_subcores=NS)


def _gather_rows(table, idx2d, group):
    nrow_idx, ch = idx2d.shape
    M = nrow_idx * ch
    D = table.shape[1]
    rpw = M // NW
    gsz = group * ch
    ngroups = rpw // gsz
    irpw = rpw // ch
    pingpong = (2 * gsz * D * 4 <= 400_000) and ngroups % 2 == 0

    nbuf = 2 if pingpong else 1

    @functools.partial(
        pl.kernel,
        out_type=jax.ShapeDtypeStruct((M, D), jnp.float32),
        mesh=_sc_mesh(),
        scratch_types=[
            pltpu.VMEM((nbuf, group, ch), jnp.int32),
            pltpu.VMEM((nbuf, gsz, D), jnp.float32),
            pltpu.SemaphoreType.DMA,
            pltpu.SemaphoreType.DMA,
        ],
    )
    def k(table_hbm, idx_hbm, out_hbm, idx_v, rows_v, sem, wsem):
        wid = lax.axis_index("s") * NC + lax.axis_index("c")
        base = wid * rpw
        ibase = wid * irpw

        def fetch(g, b):
            pltpu.sync_copy(
                idx_hbm.at[pl.ds(ibase + g * group, group)], idx_v.at[b])
            cps = [
                pltpu.async_copy(
                    table_hbm.at[idx_v.at[b, c]],
                    rows_v.at[b, pl.ds(c * ch, ch)], sem)
                for c in range(group)
            ]
            for cp in cps:
                cp.wait()

        def wb(g, b):
            return pltpu.async_copy(
                rows_v.at[b], out_hbm.at[pl.ds(base + g * gsz, gsz)], wsem)

        if pingpong:
            def body(i, _):
                g0 = 2 * i
                fetch(g0, 0)
                w0 = wb(g0, 0)
                fetch(g0 + 1, 1)
                w0.wait()
                w1 = wb(g0 + 1, 1)
                w1.wait()
                return 0

            lax.fori_loop(0, ngroups // 2, body, 0)
        else:
            def body(g, _):
                fetch(g, 0)
                wb(g, 0).wait()
                return 0

            lax.fori_loop(0, ngroups, body, 0)

    return k(table, idx2d)


NPH = 2
QTR = 4096
ACC_ROWS = QTR + 256
DEN_BASE = QTR + 128


def _step_scatter(msg, rows, drows, mseg_nc, oseg_nc):
    mm, D = msg.shape
    mo = rows.shape[0]
    spt = ACC_ROWS // NS
    gm = 4
    gsz = gm * CH

    @functools.partial(
        pl.kernel,
        out_type=jax.ShapeDtypeStruct((NC, 2 * NPH, ACC_ROWS, D), jnp.float32),
        mesh=_sc_mesh(),
        compiler_params=pltpu.CompilerParams(needs_layout_passes=False),
        scratch_types=[
            pltpu.VMEM((gm, CH), jnp.int32),
            pltpu.VMEM((gm, CH), jnp.int32),
            pltpu.VMEM((gsz, D), jnp.float32),
            pltpu.VMEM_SHARED((ACC_ROWS, D), jnp.float32),
            pltpu.SemaphoreType.DMA,
            pltpu.SemaphoreType.DMA,
        ],
    )
    def k(msg_hbm, rows_hbm, drows_hbm, mseg_hbm, oseg_hbm,
          zero_hbm, out_hbm, idx_v, didx_v, rows_v, acc, ssem, asem):
        cid = lax.axis_index("c")
        sid = lax.axis_index("s")

        def scatter_stream(vals_hbm, idx_hbm, m, ph, dvals_hbm=None):
            rpw = m // NS
            base = sid * rpw
            ibase = sid * (rpw // CH)

            def do_adds(vals, iv, g):
                pltpu.sync_copy(vals.at[pl.ds(base + g * gsz, gsz)], rows_v)
                adds = [
                    pltpu.async_copy(rows_v.at[pl.ds(c * CH, CH)],
                                     acc.at[iv.at[c]], asem, add=True)
                    for c in range(gm)
                ]
                for a in adds:
                    a.wait()

            def body(g, _):
                pltpu.sync_copy(
                    idx_hbm.at[cid, ph, pl.ds(ibase + g * gm, gm)], idx_v)
                red = idx_v[0, pl.ds(0, 16)]
                for c in range(gm):
                    for j in range(8):
                        red = jnp.minimum(red, idx_v[c, pl.ds(j * 16, 16)])
                live = lax.reduce_min(red, (0,)) < QTR

                @pl.when(live)
                def _():
                    if dvals_hbm is not None:
                        for c in range(gm):
                            for j in range(8):
                                didx_v[c, pl.ds(j * 16, 16)] = (
                                    DEN_BASE + lax.shift_right_logical(
                                        idx_v[c, pl.ds(j * 16, 16)], 7))
                    do_adds(vals_hbm, idx_v, g)
                    if dvals_hbm is not None:
                        do_adds(dvals_hbm, didx_v, g)

                return 0

            lax.fori_loop(0, rpw // gsz, body, 0)

        def zero_and(phase_fn, slot):
            pltpu.sync_copy(zero_hbm, acc.at[pl.ds(sid * spt, spt)])
            plsc.subcore_barrier()
            phase_fn()
            plsc.subcore_barrier()
            pltpu.sync_copy(acc.at[pl.ds(sid * spt, spt)],
                            out_hbm.at[cid, slot, pl.ds(sid * spt, spt)])
            plsc.subcore_barrier()

        for ph in range(NPH):
            zero_and(
                functools.partial(scatter_stream, msg_hbm, mseg_hbm, mm, ph),
                ph)
        for ph in range(NPH):
            zero_and(
                functools.partial(scatter_stream, rows_hbm, oseg_hbm, mo, ph,
                                  drows_hbm),
                NPH + ph)

    zero = jnp.zeros((spt, D), jnp.float32)
    out = k(msg, rows, drows, mseg_nc, oseg_nc, zero)
    im = jnp.stack([
        jnp.concatenate([out[c, 0, :QTR], out[c, 1, :QTR]])
        for c in range(NC)
    ])
    om = jnp.stack([
        jnp.concatenate([out[c, NPH, :QTR], out[c, NPH + 1, :QTR]])
        for c in range(NC)
    ])
    den = jnp.stack([
        jnp.concatenate([
            out[c, NPH, DEN_BASE:DEN_BASE + 32].reshape(QTR, 1),
            out[c, NPH + 1, DEN_BASE:DEN_BASE + 32].reshape(QTR, 1),
        ]) for c in range(NC)
    ])
    return im, om, den


def _segsum_split(vals, idx_nc, nseg, group):
    M, D = vals.shape
    _, nrow_idx, ch = idx_nc.shape
    half = nseg // NC
    nacc = half + 128
    rpw = M // NS
    gsz = group * ch
    ngroups = rpw // gsz
    irpw = rpw // ch
    spt = nacc // NS

    @functools.partial(
        pl.kernel,
        out_type=jax.ShapeDtypeStruct((NC, nacc, D), jnp.float32),
        mesh=_sc_mesh(),
        scratch_types=[
            pltpu.VMEM((group, ch), jnp.int32),
            pltpu.VMEM((gsz, D), jnp.float32),
            pltpu.VMEM_SHARED((nacc, D), jnp.float32),
            pltpu.SemaphoreType.DMA,
            pltpu.SemaphoreType.DMA,
        ],
    )
    def k(vals_hbm, idx_hbm, zero_hbm, out_hbm, idx_v, rows_v, acc,
          ssem, asem):
        cid = lax.axis_index("c")
        sid = lax.axis_index("s")
        base = sid * rpw
        ibase = sid * irpw

        pltpu.sync_copy(zero_hbm, acc.at[pl.ds(sid * spt, spt)])
        plsc.subcore_barrier()

        def body(g, _):
            c1 = pltpu.async_copy(
                idx_hbm.at[cid, pl.ds(ibase + g * group, group)], idx_v, ssem)
            c2 = pltpu.async_copy(
                vals_hbm.at[pl.ds(base + g * gsz, gsz)], rows_v, ssem)
            c1.wait()
            c2.wait()
            adds = [
                pltpu.async_copy(rows_v.at[pl.ds(c * ch, ch)],
                                 acc.at[idx_v.at[c]], asem, add=True)
                for c in range(group)
            ]
            for a in adds:
                a.wait()
            return 0

        lax.fori_loop(0, ngroups, body, 0)
        plsc.subcore_barrier()
        pltpu.sync_copy(
            acc.at[pl.ds(sid * spt, spt)],
            out_hbm.at[cid, pl.ds(sid * spt, spt)])

    zero = jnp.zeros((spt, D), jnp.float32)
    out = k(vals, idx_nc, zero)
    return jnp.concatenate([out[0, :half], out[1, :half]])


def _split_idx(idx, nseg):
    half = nseg // NC
    both = []
    for c in range(NC):
        both.append(jnp.where(idx // half == c, idx % half, half))
    return jnp.stack(both).reshape(NC, -1, CH)


def _leaky(x):
    return jnp.where(x >= 0, x, 0.01 * x)


def _mm(a, b, dims):
    return lax.dot_general(a, b, (dims, ((), ())),
                           preferred_element_type=jnp.float32)


def _node_init(atype_col, feat, emb_pad, proj_w, proj_b):
    _, n, _ = feat.shape
    df = feat.shape[2]
    d = emb_pad.shape[1]
    ap = emb_pad.shape[0]

    def body(t_ref, f_ref, e_ref, w_ref, b_ref, o_ref):
        tbl = _mm(e_ref[...], w_ref[:, :d], ((1,), (1,)))
        oh = (lax.broadcasted_iota(jnp.int32, (n, ap), 1)
              == t_ref[0]).astype(jnp.float32)
        a = _mm(oh, tbl, ((1,), (0,)))
        f = _mm(f_ref[0], w_ref[:, d:], ((1,), (1,)))
        o_ref[0] = a + f + b_ref[...]

    return pl.pallas_call(
        body,
        grid=(2,),
        in_specs=[
            pl.BlockSpec((1, n, 1), lambda i: (i, 0, 0)),
            pl.BlockSpec((1, n, df), lambda i: (i, 0, 0)),
            pl.BlockSpec((ap, d), lambda i: (0, 0)),
            pl.BlockSpec(proj_w.shape, lambda i: (0, 0)),
            pl.BlockSpec((1, d), lambda i: (0, 0)),
        ],
        out_specs=pl.BlockSpec((1, n, d), lambda i: (i, 0, 0)),
        out_shape=jax.ShapeDtypeStruct((2, n, d), jnp.float32),
    )(atype_col, feat, emb_pad, proj_w, proj_b)


def _edge_emb(btype_col, emb_pad):
    _, e, _ = btype_col.shape
    ap, d = emb_pad.shape

    def body(t_ref, e_ref, o_ref):
        oh = (lax.broadcasted_iota(jnp.int32, (e, ap), 1)
              == t_ref[0]).astype(jnp.float32)
        o_ref[0] = _mm(oh, e_ref[...], ((1,), (0,)))

    return pl.pallas_call(
        body,
        grid=(2,),
        in_specs=[
            pl.BlockSpec((1, e, 1), lambda i: (i, 0, 0)),
            pl.BlockSpec((ap, d), lambda i: (0, 0)),
        ],
        out_specs=pl.BlockSpec((1, e, d), lambda i: (i, 0, 0)),
        out_shape=jax.ShapeDtypeStruct((2, e, d), jnp.float32),
    )(btype_col, emb_pad)


def _qkv(node, wq, wk, wv):
    _, n, d = node.shape

    def body(n_ref, q_ref2, k_ref2, v_ref2, q_ref, kv_ref):
        x = n_ref[0]
        q = _mm(x, q_ref2[...], ((1,), (1,)))
        kk = _mm(x, k_ref2[...], ((1,), (1,)))
        vv = _mm(x, v_ref2[...], ((1,), (1,)))
        q_ref[0] = q
        kv_ref[0] = jnp.concatenate([kk, vv], axis=-1)

    return pl.pallas_call(
        body,
        grid=(2,),
        in_specs=[
            pl.BlockSpec((1, n, d), lambda i: (i, 0, 0)),
            pl.BlockSpec((d, d), lambda i: (0, 0)),
            pl.BlockSpec((d, d), lambda i: (0, 0)),
            pl.BlockSpec((d, d), lambda i: (0, 0)),
        ],
        out_specs=[
            pl.BlockSpec((1, n, d), lambda i: (i, 0, 0)),
            pl.BlockSpec((1, n, 2 * d), lambda i: (i, 0, 0)),
        ],
        out_shape=[
            jax.ShapeDtypeStruct((2, n, d), jnp.float32),
            jax.ShapeDtypeStruct((2, n, 2 * d), jnp.float32),
        ],
    )(node, wq, wk, wv)


def _msg(g, e, w_msg, b_msg):
    m, d = g.shape
    blk = 4096

    def body(g_ref, e_ref, w_ref, b_ref, o_ref):
        h = (_mm(g_ref[...], w_ref[:, :d], ((1,), (1,)))
             + _mm(e_ref[...], w_ref[:, d:], ((1,), (1,)))
             + b_ref[...])
        o_ref[...] = _leaky(h)

    return pl.pallas_call(
        body,
        grid=(m // blk,),
        in_specs=[
            pl.BlockSpec((blk, d), lambda i: (i, 0)),
            pl.BlockSpec((blk, d), lambda i: (i, 0)),
            pl.BlockSpec(w_msg.shape, lambda i: (0, 0)),
            pl.BlockSpec((1, d), lambda i: (0, 0)),
        ],
        out_specs=pl.BlockSpec((blk, d), lambda i: (i, 0)),
        out_shape=jax.ShapeDtypeStruct((m, d), jnp.float32),
    )(g, e, w_msg, b_msg)


def _att(qg, kvg, seg_col, d):
    m = qg.shape[0]
    blk = 4096
    scale = 1.0 / math.sqrt(float(d))

    def body(q_ref, kv_ref, sg_ref, o_ref, den_ref):
        q = q_ref[...]
        k = kv_ref[:, :d]
        v = kv_ref[:, d:]
        s = jnp.sum(q * k, axis=1, keepdims=True) * scale
        ex = jnp.exp(s)
        o_ref[...] = ex * v
        lane = lax.broadcasted_iota(jnp.int32, (blk, 128), 1)
        den_ref[...] = jnp.where(
            lane == lax.bitwise_and(sg_ref[...], 127), ex, 0.0)

    return pl.pallas_call(
        body,
        grid=(m // blk,),
        in_specs=[
            pl.BlockSpec((blk, d), lambda i: (i, 0)),
            pl.BlockSpec((blk, 2 * d), lambda i: (i, 0)),
            pl.BlockSpec((blk, 1), lambda i: (i, 0)),
        ],
        out_specs=[
            pl.BlockSpec((blk, d), lambda i: (i, 0)),
            pl.BlockSpec((blk, 128), lambda i: (i, 0)),
        ],
        out_shape=[
            jax.ShapeDtypeStruct((m, d), jnp.float32),
            jax.ShapeDtypeStruct((m, 128), jnp.float32),
        ],
    )(qg, kvg, seg_col)


def _upd(node, ln_g, ln_b, imp, omp, denp):
    _, n, d = node.shape

    def body(n_ref, g_ref, b_ref, im_ref, om_ref, dn_ref, o_ref):
        x = n_ref[0]
        mu = jnp.mean(x, axis=1, keepdims=True)
        xc = x - mu
        var = jnp.mean(xc * xc, axis=1, keepdims=True)
        ln = xc * lax.rsqrt(var + 1e-5) * g_ref[...] + b_ref[...]
        o_ref[0] = ln + im_ref[0] + om_ref[0] / (dn_ref[0] + 1e-9)

    return pl.pallas_call(
        body,
        grid=(2,),
        in_specs=[
            pl.BlockSpec((1, n, d), lambda i: (i, 0, 0)),
            pl.BlockSpec((1, d), lambda i: (0, 0)),
            pl.BlockSpec((1, d), lambda i: (0, 0)),
            pl.BlockSpec((1, n, d), lambda i: (i, 0, 0)),
            pl.BlockSpec((1, n, d), lambda i: (i, 0, 0)),
            pl.BlockSpec((1, n, 1), lambda i: (i, 0, 0)),
        ],
        out_specs=pl.BlockSpec((1, n, d), lambda i: (i, 0, 0)),
        out_shape=jax.ShapeDtypeStruct((2, n, d), jnp.float32),
    )(node, ln_g, ln_b, imp, omp, denp)


def _ro(node, ro_w, ro_b):
    _, n, d = node.shape

    def body(n_ref, w_ref, b_ref, o_ref):
        o_ref[0] = _leaky(_mm(n_ref[0], w_ref[...], ((1,), (1,))) + b_ref[...])

    return pl.pallas_call(
        body,
        grid=(2,),
        in_specs=[
            pl.BlockSpec((1, n, d), lambda i: (i, 0, 0)),
            pl.BlockSpec((d, d), lambda i: (0, 0)),
            pl.BlockSpec((1, d), lambda i: (0, 0)),
        ],
        out_specs=pl.BlockSpec((1, n, d), lambda i: (i, 0, 0)),
        out_shape=jax.ShapeDtypeStruct((2, n, d), jnp.float32),
    )(node, ro_w, ro_b)


def _transe(wh_r, wt_r, se_pad, dvp, nb):
    ep, d = se_pad.shape
    eb = ep // nb

    def body(wh_ref, wt_ref, se_ref, dv_ref, o_ref):
        d1 = dv_ref[0]
        d2 = dv_ref[1]
        wh = wh_ref[...]
        wt = wt_ref[...]
        ph1 = _mm(wh, d1, ((1,), (1,))).reshape(eb, d, 256)
        pt2 = _mm(wt, d2, ((1,), (1,))).reshape(eb, d, 256)
        ph2 = _mm(wh, d2, ((1,), (1,))).reshape(eb, d, 256)
        pt1 = _mm(wt, d1, ((1,), (1,))).reshape(eb, d, 256)
        r = se_ref[...]
        r3 = r.reshape(eb, d, 1)
        f3 = ph1 + r3 - pt2
        b3 = ph2 + r3 - pt1
        fwd = jnp.sqrt(jnp.sum(f3 * f3, axis=1))
        bwd = jnp.sqrt(jnp.sum(b3 * b3, axis=1))
        nh1 = jnp.sqrt(jnp.sum(ph1 * ph1, axis=1))
        nt2 = jnp.sqrt(jnp.sum(pt2 * pt2, axis=1))
        nh2 = jnp.sqrt(jnp.sum(ph2 * ph2, axis=1))
        nt1 = jnp.sqrt(jnp.sum(pt1 * pt1, axis=1))
        nse = jnp.broadcast_to(
            jnp.sqrt(jnp.sum(r * r, axis=1, keepdims=True)), (eb, 256))
        o_ref[...] = jnp.stack([fwd + bwd, nh1, nt2, nh2, nt1, nse], axis=1)

    return pl.pallas_call(
        body,
        grid=(nb,),
        in_specs=[
            pl.BlockSpec((eb * d, d), lambda i: (i, 0)),
            pl.BlockSpec((eb * d, d), lambda i: (i, 0)),
            pl.BlockSpec((eb, d), lambda i: (i, 0)),
            pl.BlockSpec(dvp.shape, lambda i: (0, 0, 0)),
        ],
        out_specs=pl.BlockSpec((eb, 6, 256), lambda i: (i, 0, 0)),
        out_shape=jax.ShapeDtypeStruct((ep, 6, 256), jnp.float32),
    )(wh_r, wt_r, se_pad, dvp)


def _final(grows, dss_col, dvp):
    s = grows.shape[0]

    def body(g_ref, ds_ref, dv_ref, sc_ref, nl_ref):
        oh = (lax.broadcasted_iota(jnp.int32, (s, 256), 1)
              == ds_ref[...]).astype(jnp.float32)
        g = g_ref[...]

        def sel(k):
            return jnp.sum(g[:, k * 256:(k + 1) * 256] * oh,
                           axis=1, keepdims=True)

        sc_ref[...] = sel(0)
        acc = jnp.float32(0.0)
        for k in range(1, 6):
            acc = acc + jnp.sum(jnp.maximum(sel(k) - 1.0, 0.0)) / s
        d1 = dv_ref[0]
        d2 = dv_ref[1]
        nd1 = jnp.sqrt(jnp.sum(d1 * d1, axis=1, keepdims=True))
        nd2 = jnp.sqrt(jnp.sum(d2 * d2, axis=1, keepdims=True))
        acc = acc + jnp.sum(jnp.maximum(nd1 - 1.0, 0.0)) / 256.0
        acc = acc + jnp.sum(jnp.maximum(nd2 - 1.0, 0.0)) / 256.0
        nl_ref[...] = jnp.reshape(acc, (1, 1))

    return pl.pallas_call(
        body,
        out_shape=[
            jax.ShapeDtypeStruct((s, 1), jnp.float32),
            jax.ShapeDtypeStruct((1, 1), jnp.float32),
        ],
    )(grows, dss_col, dvp)


def kernel(atom_feat1, atom_feat2, atom_emb, atom_proj_w, atom_proj_b,
           bond_emb, side_effect_emb, se_head_proj_w, se_tail_proj_w,
           enc_params, ro_w, ro_b, seg_m1, atom_type1, bond_type1,
           inn_seg_i1, inn_idx_j1, out_seg_i1, out_idx_j1, seg_m2,
           atom_type2, bond_type2, inn_seg_i2, inn_idx_j2, out_seg_i2,
           out_idx_j2, se_idx, drug_se_seg):
    d = atom_emb.shape[1]
    n = atom_type1.shape[0]
    e = bond_type1.shape[0]
    eo = out_seg_i1.shape[0]
    s = se_idx.shape[0]
    nse = side_effect_emb.shape[0]
    nmol = 256

    i32 = jnp.int32
    emb_pad = jnp.pad(atom_emb, ((0, 128 - atom_emb.shape[0]), (0, 0)))
    bemb_pad = jnp.pad(bond_emb, ((0, 32 - bond_emb.shape[0]), (0, 0)))
    atype = jnp.stack([atom_type1, atom_type2]).astype(i32)[:, :, None]
    feat = jnp.stack([atom_feat1, atom_feat2])
    node = _node_init(atype, feat, emb_pad, atom_proj_w, atom_proj_b[None])
    btype = jnp.stack([bond_type1, bond_type2]).astype(i32)[:, :, None]
    edge = _edge_emb(btype, bemb_pad).reshape(2 * e, d)

    half = n
    mp_idx = jnp.concatenate(
        [inn_idx_j1, inn_idx_j2 + n]).astype(i32).reshape(-1, CH)
    mseg_all = jnp.concatenate(
        [inn_seg_i1, inn_seg_i2 + n]).astype(i32)
    oseg_all = jnp.concatenate(
        [out_seg_i1, out_seg_i2 + n]).astype(i32)
    q_idx = oseg_all.reshape(-1, CH)
    kv_idx = jnp.concatenate(
        [out_idx_j1, out_idx_j2 + n]).astype(i32).reshape(-1, CH)
    o_seg_col = jnp.concatenate(
        [out_seg_i1, out_seg_i2]).astype(i32)[:, None]

    def remap(seg):
        q = seg // QTR
        return jnp.stack([
            jnp.stack([
                jnp.where(q == c * NPH + ph, seg - (c * NPH + ph) * QTR, QTR)
                for ph in range(NPH)
            ]) for c in range(NC)
        ]).reshape(NC, NPH, -1, CH)


    mseg_nc = remap(mseg_all)
    oseg_nc = remap(oseg_all)

    pstk = jax.tree.map(lambda *xs: jnp.stack(xs), *enc_params)

    def step(nd, p):
        q, kv = _qkv(nd, p['Wq'], p['Wk'], p['Wv'])
        g = _gather_rows(nd.reshape(2 * n, d), mp_idx, 2)
        msg = _msg(g, edge, p['W_msg'], p['b_msg'][None])
        qg = _gather_rows(q.reshape(2 * n, d), q_idx, 2)
        kvg = _gather_rows(kv[::-1].reshape(2 * n, 2 * d), kv_idx, 1)
        rows, drows = _att(qg, kvg, o_seg_col, d)
        im, om, den = _step_scatter(msg, rows, drows, mseg_nc, oseg_nc)
        return _upd(nd, p['ln_g'][None], p['ln_b'][None], im, om, den), 0

    node, _ = lax.scan(step, node, pstk)

    ro = _ro(node, ro_w, ro_b[None]).reshape(2 * n, d)
    ro_seg = _split_idx(
        jnp.concatenate([seg_m1, seg_m2 + nmol]).astype(i32), 2 * nmol)
    dvp = _segsum_split(ro, ro_seg, 2 * nmol, 4).reshape(2, nmol, d)

    nb = 121
    ep = nb * 8
    wh_r = jnp.pad(se_head_proj_w,
                   ((0, ep - nse), (0, 0), (0, 0))).reshape(ep * d, d)
    wt_r = jnp.pad(se_tail_proj_w,
                   ((0, ep - nse), (0, 0), (0, 0))).reshape(ep * d, d)
    se_pad = jnp.pad(side_effect_emb, ((0, ep - nse), (0, 0)))
    tbl = _transe(wh_r, wt_r, se_pad, dvp, nb).reshape(ep, 6 * 256)
    grows = _gather_rows(tbl, se_idx.astype(i32).reshape(-1, 64), 1)
    score_col, nl = _final(grows, drug_se_seg.astype(i32)[:, None], dvp)
    return score_col[:, 0], nl[0, 0]


# speedup vs baseline: 1.0314x; 1.0314x over previous
import functools
import math

import jax
import jax.numpy as jnp
from jax import lax
from jax.experimental import pallas as pl
from jax.experimental.pallas import tpu as pltpu
from jax.experimental.pallas import tpu_sc as plsc

NC, NS, CH = 2, 16, 128
NW = NC * NS


def _sc_mesh():
    return plsc.VectorSubcoreMesh(
        core_axis_name="c", subcore_axis_name="s", num_cores=NC, num_subcores=NS)


def _gather_rows(table, idx2d, group):
    nrow_idx, ch = idx2d.shape
    M = nrow_idx * ch
    D = table.shape[1]
    rpw = M // NW
    gsz = group * ch
    ngroups = rpw // gsz
    irpw = rpw // ch

    @functools.partial(
        pl.kernel,
        out_type=jax.ShapeDtypeStruct((M, D), jnp.float32),
        mesh=_sc_mesh(),
        scratch_types=[
            pltpu.VMEM((group, ch), jnp.int32),
            pltpu.VMEM((gsz, D), jnp.float32),
            pltpu.SemaphoreType.DMA,
        ],
    )
    def k(table_hbm, idx_hbm, out_hbm, idx_v, rows_v, sem):
        wid = lax.axis_index("s") * NC + lax.axis_index("c")
        base = wid * rpw
        ibase = wid * irpw

        def body(g, _):
            pltpu.sync_copy(idx_hbm.at[pl.ds(ibase + g * group, group)], idx_v)
            cps = [
                pltpu.async_copy(
                    table_hbm.at[idx_v.at[c]],
                    rows_v.at[pl.ds(c * ch, ch)], sem)
                for c in range(group)
            ]
            for cp in cps:
                cp.wait()
            pltpu.sync_copy(rows_v, out_hbm.at[pl.ds(base + g * gsz, gsz)])
            return 0

        lax.fori_loop(0, ngroups, body, 0)

    return k(table, idx2d)


NPH = 2
QTR = 4096
ACC_ROWS = QTR + 256
DEN_BASE = QTR + 128


def _step_scatter(msg, rows, drows, mseg_nc, oseg_nc):
    mm, D = msg.shape
    mo = rows.shape[0]
    spt = ACC_ROWS // NS
    gm = 4
    gsz = gm * CH

    @functools.partial(
        pl.kernel,
        out_type=jax.ShapeDtypeStruct((NC, 2 * NPH, ACC_ROWS, D), jnp.float32),
        mesh=_sc_mesh(),
        compiler_params=pltpu.CompilerParams(needs_layout_passes=False),
        scratch_types=[
            pltpu.VMEM((gm, CH), jnp.int32),
            pltpu.VMEM((gm, CH), jnp.int32),
            pltpu.VMEM((gsz, D), jnp.float32),
            pltpu.VMEM_SHARED((ACC_ROWS, D), jnp.float32),
            pltpu.SemaphoreType.DMA,
            pltpu.SemaphoreType.DMA,
        ],
    )
    def k(msg_hbm, rows_hbm, drows_hbm, mseg_hbm, oseg_hbm,
          zero_hbm, out_hbm, idx_v, didx_v, rows_v, acc, ssem, asem):
        cid = lax.axis_index("c")
        sid = lax.axis_index("s")

        def scatter_stream(vals_hbm, idx_hbm, m, ph, dvals_hbm=None):
            rpw = m // NS
            base = sid * rpw
            ibase = sid * (rpw // CH)

            def do_adds(vals, iv, g):
                pltpu.sync_copy(vals.at[pl.ds(base + g * gsz, gsz)], rows_v)
                adds = [
                    pltpu.async_copy(rows_v.at[pl.ds(c * CH, CH)],
                                     acc.at[iv.at[c]], asem, add=True)
                    for c in range(gm)
                ]
                for a in adds:
                    a.wait()

            def body(g, _):
                pltpu.sync_copy(
                    idx_hbm.at[cid, ph, pl.ds(ibase + g * gm, gm)], idx_v)
                red = idx_v[0, pl.ds(0, 16)]
                for c in range(gm):
                    for j in range(8):
                        red = jnp.minimum(red, idx_v[c, pl.ds(j * 16, 16)])
                live = lax.reduce_min(red, (0,)) < QTR

                @pl.when(live)
                def _():
                    if dvals_hbm is not None:
                        for c in range(gm):
                            for j in range(8):
                                didx_v[c, pl.ds(j * 16, 16)] = (
                                    DEN_BASE + lax.shift_right_logical(
                                        idx_v[c, pl.ds(j * 16, 16)], 7))
                    do_adds(vals_hbm, idx_v, g)
                    if dvals_hbm is not None:
                        do_adds(dvals_hbm, didx_v, g)

                return 0

            lax.fori_loop(0, rpw // gsz, body, 0)

        def zero_and(phase_fn, slot):
            pltpu.sync_copy(zero_hbm, acc.at[pl.ds(sid * spt, spt)])
            plsc.subcore_barrier()
            phase_fn()
            plsc.subcore_barrier()
            pltpu.sync_copy(acc.at[pl.ds(sid * spt, spt)],
                            out_hbm.at[cid, slot, pl.ds(sid * spt, spt)])
            plsc.subcore_barrier()

        for ph in range(NPH):
            zero_and(
                functools.partial(scatter_stream, msg_hbm, mseg_hbm, mm, ph),
                ph)
        for ph in range(NPH):
            zero_and(
                functools.partial(scatter_stream, rows_hbm, oseg_hbm, mo, ph,
                                  drows_hbm),
                NPH + ph)

    zero = jnp.zeros((spt, D), jnp.float32)
    out = k(msg, rows, drows, mseg_nc, oseg_nc, zero)
    im = jnp.stack([
        jnp.concatenate([out[c, 0, :QTR], out[c, 1, :QTR]])
        for c in range(NC)
    ])
    om = jnp.stack([
        jnp.concatenate([out[c, NPH, :QTR], out[c, NPH + 1, :QTR]])
        for c in range(NC)
    ])
    den = jnp.stack([
        jnp.concatenate([
            out[c, NPH, DEN_BASE:DEN_BASE + 32].reshape(QTR, 1),
            out[c, NPH + 1, DEN_BASE:DEN_BASE + 32].reshape(QTR, 1),
        ]) for c in range(NC)
    ])
    return im, om, den


def _segsum_split(vals, idx_nc, nseg, group):
    M, D = vals.shape
    _, nrow_idx, ch = idx_nc.shape
    half = nseg // NC
    nacc = half + 128
    rpw = M // NS
    gsz = group * ch
    ngroups = rpw // gsz
    irpw = rpw // ch
    spt = nacc // NS

    @functools.partial(
        pl.kernel,
        out_type=jax.ShapeDtypeStruct((NC, nacc, D), jnp.float32),
        mesh=_sc_mesh(),
        scratch_types=[
            pltpu.VMEM((group, ch), jnp.int32),
            pltpu.VMEM((gsz, D), jnp.float32),
            pltpu.VMEM_SHARED((nacc, D), jnp.float32),
            pltpu.SemaphoreType.DMA,
            pltpu.SemaphoreType.DMA,
        ],
    )
    def k(vals_hbm, idx_hbm, zero_hbm, out_hbm, idx_v, rows_v, acc,
          ssem, asem):
        cid = lax.axis_index("c")
        sid = lax.axis_index("s")
        base = sid * rpw
        ibase = sid * irpw

        pltpu.sync_copy(zero_hbm, acc.at[pl.ds(sid * spt, spt)])
        plsc.subcore_barrier()

        def body(g, _):
            c1 = pltpu.async_copy(
                idx_hbm.at[cid, pl.ds(ibase + g * group, group)], idx_v, ssem)
            c2 = pltpu.async_copy(
                vals_hbm.at[pl.ds(base + g * gsz, gsz)], rows_v, ssem)
            c1.wait()
            c2.wait()
            adds = [
                pltpu.async_copy(rows_v.at[pl.ds(c * ch, ch)],
                                 acc.at[idx_v.at[c]], asem, add=True)
                for c in range(group)
            ]
            for a in adds:
                a.wait()
            return 0

        lax.fori_loop(0, ngroups, body, 0)
        plsc.subcore_barrier()
        pltpu.sync_copy(
            acc.at[pl.ds(sid * spt, spt)],
            out_hbm.at[cid, pl.ds(sid * spt, spt)])

    zero = jnp.zeros((spt, D), jnp.float32)
    out = k(vals, idx_nc, zero)
    return jnp.concatenate([out[0, :half], out[1, :half]])


def _split_idx(idx, nseg):
    half = nseg // NC
    both = []
    for c in range(NC):
        both.append(jnp.where(idx // half == c, idx % half, half))
    return jnp.stack(both).reshape(NC, -1, CH)


def _leaky(x):
    return jnp.where(x >= 0, x, 0.01 * x)


def _mm(a, b, dims):
    return lax.dot_general(a, b, (dims, ((), ())),
                           preferred_element_type=jnp.float32)


def _node_init(atype_col, feat, emb_pad, proj_w, proj_b):
    _, n, _ = feat.shape
    df = feat.shape[2]
    d = emb_pad.shape[1]
    ap = emb_pad.shape[0]

    def body(t_ref, f_ref, e_ref, w_ref, b_ref, o_ref):
        tbl = _mm(e_ref[...], w_ref[:, :d], ((1,), (1,)))
        oh = (lax.broadcasted_iota(jnp.int32, (n, ap), 1)
              == t_ref[0]).astype(jnp.float32)
        a = _mm(oh, tbl, ((1,), (0,)))
        f = _mm(f_ref[0], w_ref[:, d:], ((1,), (1,)))
        o_ref[0] = a + f + b_ref[...]

    return pl.pallas_call(
        body,
        grid=(2,),
        in_specs=[
            pl.BlockSpec((1, n, 1), lambda i: (i, 0, 0)),
            pl.BlockSpec((1, n, df), lambda i: (i, 0, 0)),
            pl.BlockSpec((ap, d), lambda i: (0, 0)),
            pl.BlockSpec(proj_w.shape, lambda i: (0, 0)),
            pl.BlockSpec((1, d), lambda i: (0, 0)),
        ],
        out_specs=pl.BlockSpec((1, n, d), lambda i: (i, 0, 0)),
        out_shape=jax.ShapeDtypeStruct((2, n, d), jnp.float32),
    )(atype_col, feat, emb_pad, proj_w, proj_b)


def _edge_emb(btype_col, emb_pad):
    _, e, _ = btype_col.shape
    ap, d = emb_pad.shape

    def body(t_ref, e_ref, o_ref):
        oh = (lax.broadcasted_iota(jnp.int32, (e, ap), 1)
              == t_ref[0]).astype(jnp.float32)
        o_ref[0] = _mm(oh, e_ref[...], ((1,), (0,)))

    return pl.pallas_call(
        body,
        grid=(2,),
        in_specs=[
            pl.BlockSpec((1, e, 1), lambda i: (i, 0, 0)),
            pl.BlockSpec((ap, d), lambda i: (0, 0)),
        ],
        out_specs=pl.BlockSpec((1, e, d), lambda i: (i, 0, 0)),
        out_shape=jax.ShapeDtypeStruct((2, e, d), jnp.float32),
    )(btype_col, emb_pad)


def _qkv(node, wq, wk, wv):
    _, n, d = node.shape

    def body(n_ref, q_ref2, k_ref2, v_ref2, q_ref, kv_ref):
        x = n_ref[0]
        q = _mm(x, q_ref2[...], ((1,), (1,)))
        kk = _mm(x, k_ref2[...], ((1,), (1,)))
        vv = _mm(x, v_ref2[...], ((1,), (1,)))
        q_ref[0] = q
        kv_ref[0] = jnp.concatenate([kk, vv], axis=-1)

    return pl.pallas_call(
        body,
        grid=(2,),
        in_specs=[
            pl.BlockSpec((1, n, d), lambda i: (i, 0, 0)),
            pl.BlockSpec((d, d), lambda i: (0, 0)),
            pl.BlockSpec((d, d), lambda i: (0, 0)),
            pl.BlockSpec((d, d), lambda i: (0, 0)),
        ],
        out_specs=[
            pl.BlockSpec((1, n, d), lambda i: (i, 0, 0)),
            pl.BlockSpec((1, n, 2 * d), lambda i: (i, 0, 0)),
        ],
        out_shape=[
            jax.ShapeDtypeStruct((2, n, d), jnp.float32),
            jax.ShapeDtypeStruct((2, n, 2 * d), jnp.float32),
        ],
    )(node, wq, wk, wv)


def _msg(g, e, w_msg, b_msg):
    m, d = g.shape
    blk = 4096

    def body(g_ref, e_ref, w_ref, b_ref, o_ref):
        h = (_mm(g_ref[...], w_ref[:, :d], ((1,), (1,)))
             + _mm(e_ref[...], w_ref[:, d:], ((1,), (1,)))
             + b_ref[...])
        o_ref[...] = _leaky(h)

    return pl.pallas_call(
        body,
        grid=(m // blk,),
        in_specs=[
            pl.BlockSpec((blk, d), lambda i: (i, 0)),
            pl.BlockSpec((blk, d), lambda i: (i, 0)),
            pl.BlockSpec(w_msg.shape, lambda i: (0, 0)),
            pl.BlockSpec((1, d), lambda i: (0, 0)),
        ],
        out_specs=pl.BlockSpec((blk, d), lambda i: (i, 0)),
        out_shape=jax.ShapeDtypeStruct((m, d), jnp.float32),
    )(g, e, w_msg, b_msg)


def _att(qg, kvg, seg_col, d):
    m = qg.shape[0]
    blk = 4096
    scale = 1.0 / math.sqrt(float(d))

    def body(q_ref, kv_ref, sg_ref, o_ref, den_ref):
        q = q_ref[...]
        k = kv_ref[:, :d]
        v = kv_ref[:, d:]
        s = jnp.sum(q * k, axis=1, keepdims=True) * scale
        ex = jnp.exp(s)
        o_ref[...] = ex * v
        lane = lax.broadcasted_iota(jnp.int32, (blk, 128), 1)
        den_ref[...] = jnp.where(
            lane == lax.bitwise_and(sg_ref[...], 127), ex, 0.0)

    return pl.pallas_call(
        body,
        grid=(m // blk,),
        in_specs=[
            pl.BlockSpec((blk, d), lambda i: (i, 0)),
            pl.BlockSpec((blk, 2 * d), lambda i: (i, 0)),
            pl.BlockSpec((blk, 1), lambda i: (i, 0)),
        ],
        out_specs=[
            pl.BlockSpec((blk, d), lambda i: (i, 0)),
            pl.BlockSpec((blk, 128), lambda i: (i, 0)),
        ],
        out_shape=[
            jax.ShapeDtypeStruct((m, d), jnp.float32),
            jax.ShapeDtypeStruct((m, 128), jnp.float32),
        ],
    )(qg, kvg, seg_col)


def _upd(node, ln_g, ln_b, imp, omp, denp):
    _, n, d = node.shape

    def body(n_ref, g_ref, b_ref, im_ref, om_ref, dn_ref, o_ref):
        x = n_ref[0]
        mu = jnp.mean(x, axis=1, keepdims=True)
        xc = x - mu
        var = jnp.mean(xc * xc, axis=1, keepdims=True)
        ln = xc * lax.rsqrt(var + 1e-5) * g_ref[...] + b_ref[...]
        o_ref[0] = ln + im_ref[0] + om_ref[0] / (dn_ref[0] + 1e-9)

    return pl.pallas_call(
        body,
        grid=(2,),
        in_specs=[
            pl.BlockSpec((1, n, d), lambda i: (i, 0, 0)),
            pl.BlockSpec((1, d), lambda i: (0, 0)),
            pl.BlockSpec((1, d), lambda i: (0, 0)),
            pl.BlockSpec((1, n, d), lambda i: (i, 0, 0)),
            pl.BlockSpec((1, n, d), lambda i: (i, 0, 0)),
            pl.BlockSpec((1, n, 1), lambda i: (i, 0, 0)),
        ],
        out_specs=pl.BlockSpec((1, n, d), lambda i: (i, 0, 0)),
        out_shape=jax.ShapeDtypeStruct((2, n, d), jnp.float32),
    )(node, ln_g, ln_b, imp, omp, denp)


def _ro(node, ro_w, ro_b):
    _, n, d = node.shape

    def body(n_ref, w_ref, b_ref, o_ref):
        o_ref[0] = _leaky(_mm(n_ref[0], w_ref[...], ((1,), (1,))) + b_ref[...])

    return pl.pallas_call(
        body,
        grid=(2,),
        in_specs=[
            pl.BlockSpec((1, n, d), lambda i: (i, 0, 0)),
            pl.BlockSpec((d, d), lambda i: (0, 0)),
            pl.BlockSpec((1, d), lambda i: (0, 0)),
        ],
        out_specs=pl.BlockSpec((1, n, d), lambda i: (i, 0, 0)),
        out_shape=jax.ShapeDtypeStruct((2, n, d), jnp.float32),
    )(node, ro_w, ro_b)


def _transe(wh_r, wt_r, se_pad, dvp, nb):
    ep, d = se_pad.shape
    eb = ep // nb

    def body(wh_ref, wt_ref, se_ref, dv_ref, o_ref):
        d1 = dv_ref[0]
        d2 = dv_ref[1]
        wh = wh_ref[...]
        wt = wt_ref[...]
        ph1 = _mm(wh, d1, ((1,), (1,))).reshape(eb, d, 256)
        pt2 = _mm(wt, d2, ((1,), (1,))).reshape(eb, d, 256)
        ph2 = _mm(wh, d2, ((1,), (1,))).reshape(eb, d, 256)
        pt1 = _mm(wt, d1, ((1,), (1,))).reshape(eb, d, 256)
        r = se_ref[...]
        r3 = r.reshape(eb, d, 1)
        f3 = ph1 + r3 - pt2
        b3 = ph2 + r3 - pt1
        fwd = jnp.sqrt(jnp.sum(f3 * f3, axis=1))
        bwd = jnp.sqrt(jnp.sum(b3 * b3, axis=1))
        nh1 = jnp.sqrt(jnp.sum(ph1 * ph1, axis=1))
        nt2 = jnp.sqrt(jnp.sum(pt2 * pt2, axis=1))
        nh2 = jnp.sqrt(jnp.sum(ph2 * ph2, axis=1))
        nt1 = jnp.sqrt(jnp.sum(pt1 * pt1, axis=1))
        nse = jnp.broadcast_to(
            jnp.sqrt(jnp.sum(r * r, axis=1, keepdims=True)), (eb, 256))
        o_ref[...] = jnp.stack([fwd + bwd, nh1, nt2, nh2, nt1, nse], axis=1)

    return pl.pallas_call(
        body,
        grid=(nb,),
        in_specs=[
            pl.BlockSpec((eb * d, d), lambda i: (i, 0)),
            pl.BlockSpec((eb * d, d), lambda i: (i, 0)),
            pl.BlockSpec((eb, d), lambda i: (i, 0)),
            pl.BlockSpec(dvp.shape, lambda i: (0, 0, 0)),
        ],
        out_specs=pl.BlockSpec((eb, 6, 256), lambda i: (i, 0, 0)),
        out_shape=jax.ShapeDtypeStruct((ep, 6, 256), jnp.float32),
    )(wh_r, wt_r, se_pad, dvp)


def _final(grows, dss_col, dvp):
    s = grows.shape[0]

    def body(g_ref, ds_ref, dv_ref, sc_ref, nl_ref):
        oh = (lax.broadcasted_iota(jnp.int32, (s, 256), 1)
              == ds_ref[...]).astype(jnp.float32)
        g = g_ref[...]

        def sel(k):
            return jnp.sum(g[:, k * 256:(k + 1) * 256] * oh,
                           axis=1, keepdims=True)

        sc_ref[...] = sel(0)
        acc = jnp.float32(0.0)
        for k in range(1, 6):
            acc = acc + jnp.sum(jnp.maximum(sel(k) - 1.0, 0.0)) / s
        d1 = dv_ref[0]
        d2 = dv_ref[1]
        nd1 = jnp.sqrt(jnp.sum(d1 * d1, axis=1, keepdims=True))
        nd2 = jnp.sqrt(jnp.sum(d2 * d2, axis=1, keepdims=True))
        acc = acc + jnp.sum(jnp.maximum(nd1 - 1.0, 0.0)) / 256.0
        acc = acc + jnp.sum(jnp.maximum(nd2 - 1.0, 0.0)) / 256.0
        nl_ref[...] = jnp.reshape(acc, (1, 1))

    return pl.pallas_call(
        body,
        out_shape=[
            jax.ShapeDtypeStruct((s, 1), jnp.float32),
            jax.ShapeDtypeStruct((1, 1), jnp.float32),
        ],
    )(grows, dss_col, dvp)


def kernel(atom_feat1, atom_feat2, atom_emb, atom_proj_w, atom_proj_b,
           bond_emb, side_effect_emb, se_head_proj_w, se_tail_proj_w,
           enc_params, ro_w, ro_b, seg_m1, atom_type1, bond_type1,
           inn_seg_i1, inn_idx_j1, out_seg_i1, out_idx_j1, seg_m2,
           atom_type2, bond_type2, inn_seg_i2, inn_idx_j2, out_seg_i2,
           out_idx_j2, se_idx, drug_se_seg):
    d = atom_emb.shape[1]
    n = atom_type1.shape[0]
    e = bond_type1.shape[0]
    eo = out_seg_i1.shape[0]
    s = se_idx.shape[0]
    nse = side_effect_emb.shape[0]
    nmol = 256

    i32 = jnp.int32
    emb_pad = jnp.pad(atom_emb, ((0, 128 - atom_emb.shape[0]), (0, 0)))
    bemb_pad = jnp.pad(bond_emb, ((0, 32 - bond_emb.shape[0]), (0, 0)))
    atype = jnp.stack([atom_type1, atom_type2]).astype(i32)[:, :, None]
    feat = jnp.stack([atom_feat1, atom_feat2])
    node = _node_init(atype, feat, emb_pad, atom_proj_w, atom_proj_b[None])
    btype = jnp.stack([bond_type1, bond_type2]).astype(i32)[:, :, None]
    edge = _edge_emb(btype, bemb_pad).reshape(2 * e, d)

    half = n
    mp_idx = jnp.concatenate(
        [inn_idx_j1, inn_idx_j2 + n]).astype(i32).reshape(-1, CH)
    mseg_all = jnp.concatenate(
        [inn_seg_i1, inn_seg_i2 + n]).astype(i32)
    oseg_all = jnp.concatenate(
        [out_seg_i1, out_seg_i2 + n]).astype(i32)
    q_idx = oseg_all.reshape(-1, CH)
    kv_idx = jnp.concatenate(
        [out_idx_j1, out_idx_j2 + n]).astype(i32).reshape(-1, CH)
    o_seg_col = jnp.concatenate(
        [out_seg_i1, out_seg_i2]).astype(i32)[:, None]

    def remap(seg):
        q = seg // QTR
        return jnp.stack([
            jnp.stack([
                jnp.where(q == c * NPH + ph, seg - (c * NPH + ph) * QTR, QTR)
                for ph in range(NPH)
            ]) for c in range(NC)
        ]).reshape(NC, NPH, -1, CH)


    mseg_nc = remap(mseg_all)
    oseg_nc = remap(oseg_all)

    pstk = jax.tree.map(lambda *xs: jnp.stack(xs), *enc_params)

    def step(nd, p):
        q, kv = _qkv(nd, p['Wq'], p['Wk'], p['Wv'])
        g = _gather_rows(nd.reshape(2 * n, d), mp_idx, 4)
        msg = _msg(g, edge, p['W_msg'], p['b_msg'][None])
        qg = _gather_rows(q.reshape(2 * n, d), q_idx, 4)
        kvg = _gather_rows(kv[::-1].reshape(2 * n, 2 * d), kv_idx, 2)
        rows, drows = _att(qg, kvg, o_seg_col, d)
        im, om, den = _step_scatter(msg, rows, drows, mseg_nc, oseg_nc)
        return _upd(nd, p['ln_g'][None], p['ln_b'][None], im, om, den), 0

    node, _ = lax.scan(step, node, pstk)

    ro = _ro(node, ro_w, ro_b[None]).reshape(2 * n, d)
    ro_seg = _split_idx(
        jnp.concatenate([seg_m1, seg_m2 + nmol]).astype(i32), 2 * nmol)
    dvp = _segsum_split(ro, ro_seg, 2 * nmol, 4).reshape(2, nmol, d)

    nb = 121
    ep = nb * 8
    wh_r = jnp.pad(se_head_proj_w,
                   ((0, ep - nse), (0, 0), (0, 0))).reshape(ep * d, d)
    wt_r = jnp.pad(se_tail_proj_w,
                   ((0, ep - nse), (0, 0), (0, 0))).reshape(ep * d, d)
    se_pad = jnp.pad(side_effect_emb, ((0, ep - nse), (0, 0)))
    tbl = _transe(wh_r, wt_r, se_pad, dvp, nb).reshape(ep, 6 * 256)
    grows = _gather_rows(tbl, se_idx.astype(i32).reshape(-1, 64), 1)
    score_col, nl = _final(grows, drug_se_seg.astype(i32)[:, None], dvp)
    return score_col[:, 0], nl[0, 0]
